# baseline (speedup 1.0000x reference)
import jax
import jax.numpy as jnp
from jax.experimental import pallas as pl
from jax.experimental.pallas import tpu as pltpu

_N, _T, _D = 4, 8192, 1024
_BT = 2048
_NT = _T // _BT
_NC = _NT * _N


def _manual_kernel(x_ref, pos_ref, o_ref, xb, pb, ob, sx, sp, so):
    def x_src(c):
        t, n = divmod(c, _N)
        return x_ref.at[n, pl.ds(t * _BT, _BT), :]

    def p_src(t):
        return pos_ref.at[pl.ds(t * _BT, _BT), :]

    def o_dst(c):
        t, n = divmod(c, _N)
        return o_ref.at[n, pl.ds(t * _BT, _BT), :]

    pltpu.make_async_copy(x_src(0), xb.at[0], sx.at[0]).start()
    pltpu.make_async_copy(x_src(1), xb.at[1], sx.at[1]).start()
    pltpu.make_async_copy(p_src(0), pb.at[0], sp.at[0]).start()

    for c in range(_NC):
        b = c % 2
        t, n = divmod(c, _N)
        ts = t % 2
        if n == 0:
            pltpu.make_async_copy(p_src(t), pb.at[ts], sp.at[ts]).wait()
            if t + 1 < _NT:
                pltpu.make_async_copy(
                    p_src(t + 1), pb.at[1 - ts], sp.at[1 - ts]).start()
        pltpu.make_async_copy(x_src(c), xb.at[b], sx.at[b]).wait()
        if c >= 2:
            pltpu.make_async_copy(ob.at[b], o_dst(c - 2), so.at[b]).wait()
        ob[b, :, :] = xb[b, :, :] + pb[ts, :, :]
        pltpu.make_async_copy(ob.at[b], o_dst(c), so.at[b]).start()
        if c + 2 < _NC:
            pltpu.make_async_copy(x_src(c + 2), xb.at[b], sx.at[b]).start()

    for c in range(_NC - 2, _NC):
        b = c % 2
        pltpu.make_async_copy(ob.at[b], o_dst(c), so.at[b]).wait()


def kernel(X, pos_table):
    N, T, D = X.shape
    return pl.pallas_call(
        _manual_kernel,
        in_specs=[
            pl.BlockSpec(memory_space=pl.ANY),
            pl.BlockSpec(memory_space=pl.ANY),
        ],
        out_specs=pl.BlockSpec(memory_space=pl.ANY),
        out_shape=jax.ShapeDtypeStruct((N, T, D), X.dtype),
        scratch_shapes=[
            pltpu.VMEM((2, _BT, _D), jnp.float32),
            pltpu.VMEM((2, _BT, _D), jnp.float32),
            pltpu.VMEM((2, _BT, _D), jnp.float32),
            pltpu.SemaphoreType.DMA((2,)),
            pltpu.SemaphoreType.DMA((2,)),
            pltpu.SemaphoreType.DMA((2,)),
        ],
    )(X, pos_table)
